# baseline (speedup 1.0000x reference)
import jax
import jax.numpy as jnp
from jax.experimental import pallas as pl


def _body(x_ref, m_ref, w_ref, o_ref):
    xb = x_ref[...]
    p = jnp.dot(xb, w_ref[...], preferred_element_type=jnp.float32)
    o_ref[...] = jnp.where(m_ref[...] > 0, p, xb)


def kernel(x, mask, deletion_weight):
    n, d = x.shape
    blk = 2000
    m2 = mask.astype(jnp.int32).reshape(n, 1)
    return pl.pallas_call(
        _body,
        grid=(n // blk,),
        in_specs=[
            pl.BlockSpec((blk, d), lambda i: (i, 0)),
            pl.BlockSpec((blk, 1), lambda i: (i, 0)),
            pl.BlockSpec((d, d), lambda i: (0, 0)),
        ],
        out_specs=pl.BlockSpec((blk, d), lambda i: (i, 0)),
        out_shape=jax.ShapeDtypeStruct((n, d), x.dtype),
    )(x, m2, deletion_weight)
